# baseline (speedup 1.0000x reference)
import functools
import jax
import jax.numpy as jnp
from jax import lax
from jax.experimental import pallas as pl
from jax.experimental.pallas import tpu as pltpu
from jax.experimental.pallas import tpu_sc as plsc

B, N, M, D = 8, 2048, 8, 128
KT, KV, KE = 64, 128, 256
K_ALL = KT + KV + KE
MS = 3 * M
RW = D + 16
TM = 128
NT = N // TM
NWORK = 32
TPW = (B * N) // NWORK
CHUNK = 128
NCH = TPW // CHUNK


def _assemble_qlin(p):
    r, i, j, k = p['r'], p['i'], p['j'], p['k']
    W = jnp.concatenate([
        jnp.concatenate([r, -i, -j, -k], axis=1),
        jnp.concatenate([i, r, -k, j], axis=1),
        jnp.concatenate([j, k, r, -i], axis=1),
        jnp.concatenate([k, -j, i, r], axis=1)], axis=0)
    return W.T, p['b']


NB = 4
TPW2 = (NB * N) // NWORK
NCH2 = TPW2 // CHUNK
WPB = NWORK // NB


def _sc_scatter_body(q_hbm, idx_hbm, w_hbm, out_hbm, accf, qbuf, ibuf, wbuf):
    wid = lax.axis_index("c") * 16 + lax.axis_index("s")
    tok0 = wid * TPW2
    iota = lax.iota(jnp.int32, 16)
    lane0 = iota == 0

    @plsc.parallel_loop(0, (K_ALL * RW) // 16, unroll=8)
    def _zero(i):
        accf[pl.ds(i * 16, 16)] = jnp.zeros((16,), jnp.float32)

    for ch in range(NCH2):
        base = tok0 + ch * CHUNK
        pltpu.sync_copy(q_hbm.at[pl.ds(base * D, CHUNK * D)], qbuf)
        pltpu.sync_copy(idx_hbm.at[pl.ds(base * MS, CHUNK * MS)], ibuf)
        pltpu.sync_copy(w_hbm.at[pl.ds(base * MS, CHUNK * MS)], wbuf)

        @plsc.parallel_loop(0, CHUNK, unroll=3)
        def _tok(t):
            qv = [qbuf[pl.ds(t * D + cb * 16, 16)] for cb in range(D // 16)]
            for j in range(MS):
                sp = jnp.full((16,), t * MS + j, jnp.int32)
                rowv = plsc.load_gather(ibuf, [sp])
                wv = plsc.load_gather(wbuf, [sp])
                addr = rowv * RW + iota
                for cb in range(D // 16):
                    plsc.addupdate_scatter(accf, [addr + cb * 16], qv[cb] * wv)
                plsc.addupdate_scatter(accf, [addr + D], wv, mask=lane0)

    pltpu.sync_copy(accf, out_hbm.at[pl.ds(wid * K_ALL * RW, K_ALL * RW)])


def _sc_scatter(q_half, idx_half, w_half):
    mesh = plsc.VectorSubcoreMesh(core_axis_name="c", subcore_axis_name="s")
    f = functools.partial(
        pl.kernel, mesh=mesh,
        out_type=jax.ShapeDtypeStruct((NWORK * K_ALL * RW,), jnp.float32),
        scratch_types=[
            pltpu.VMEM((K_ALL * RW,), jnp.float32),
            pltpu.VMEM((CHUNK * D,), jnp.float32),
            pltpu.VMEM((CHUNK * MS,), jnp.int32),
            pltpu.VMEM((CHUNK * MS,), jnp.float32),
        ],
        compiler_params=pltpu.CompilerParams(needs_layout_passes=False),
    )(_sc_scatter_body)
    return f(q_half, idx_half, w_half)


def _mid_body(parts_ref, codes_ref, wq_ref, bq_ref, pb_ref):
    psum = jnp.sum(parts_ref[...], axis=0)
    proto = psum[:, :D]
    ws = psum[:, D:D+1]
    codes = codes_ref[...]
    wsc = jnp.maximum(ws, 0.001)
    blend = jnp.clip(ws / (ws + 0.5), 0.0, 1.0)
    pb = blend * (proto / wsc) + (1.0 - blend) * codes

    outs = []
    offs = [0, KT, KT + KV, K_ALL]
    for c in range(3):
        seg = pb[offs[c]:offs[c+1], :]
        outs.append(jnp.dot(seg, wq_ref[c], preferred_element_type=jnp.float32)
                    + bq_ref[c][None, :])
    pb_ref[0] = jnp.concatenate(outs, axis=0)


def _mid_call(parts, codes_all, wq_stack, bq_stack):
    return pl.pallas_call(
        _mid_body,
        grid=(NB,),
        in_specs=[
            pl.BlockSpec((WPB, K_ALL, RW), lambda b: (b, 0, 0)),
            pl.BlockSpec((K_ALL, D), lambda b: (0, 0)),
            pl.BlockSpec((3, D, D), lambda b: (0, 0, 0)),
            pl.BlockSpec((3, D), lambda b: (0, 0)),
        ],
        out_specs=pl.BlockSpec((1, K_ALL, D), lambda b: (b, 0, 0)),
        out_shape=jax.ShapeDtypeStruct((NB, K_ALL, D), jnp.float32),
    )(parts, codes_all, wq_stack, bq_stack)


def _sc_gather_body(pb_hbm, idx_hbm, w_hbm, g_hbm, tbuf, gbuf, ibuf, wbuf):
    wid = lax.axis_index("c") * 16 + lax.axis_index("s")
    b = wid // WPB
    tok0 = wid * TPW2
    iota = lax.iota(jnp.int32, 16)
    iota2 = iota * 2
    DH = D // 2

    pltpu.sync_copy(pb_hbm.at[pl.ds(b * K_ALL * DH, K_ALL * DH)], tbuf)

    for ch in range(NCH2):
        base = tok0 + ch * CHUNK
        pltpu.sync_copy(idx_hbm.at[pl.ds(base * MS, CHUNK * MS)], ibuf)
        pltpu.sync_copy(w_hbm.at[pl.ds(base * MS, CHUNK * MS)], wbuf)

        def _tok(t, _):
            acc_e = [jnp.zeros((16,), jnp.float32) for _ in range(DH // 16)]
            acc_o = [jnp.zeros((16,), jnp.float32) for _ in range(DH // 16)]
            for j in range(MS):
                sp = jnp.full((16,), t * MS + j, jnp.int32)
                rowv = plsc.load_gather(ibuf, [sp])
                wv = plsc.load_gather(wbuf, [sp])
                addr = rowv * DH + iota
                for cb in range(DH // 16):
                    v = plsc.load_gather(tbuf, [addr + cb * 16])
                    ev, ov = plsc.unpack(plsc.bitcast(v, jnp.bfloat16),
                                         format=plsc.PackFormat.INTERLEAVED)
                    acc_e[cb] = acc_e[cb] + ev * wv
                    acc_o[cb] = acc_o[cb] + ov * wv
            for cb in range(DH // 16):
                sbase = jnp.full((16,), t * D + cb * 32, jnp.int32) + iota2
                plsc.store_scatter(gbuf, [sbase], acc_e[cb])
                plsc.store_scatter(gbuf, [sbase + 1], acc_o[cb])
            return _
        lax.fori_loop(0, CHUNK, _tok, None)

        pltpu.sync_copy(gbuf, g_hbm.at[pl.ds(base * D, CHUNK * D)])


def _sc_gather(pbp_flat, idx_flat, w_flat):
    mesh = plsc.VectorSubcoreMesh(core_axis_name="c", subcore_axis_name="s")
    f = functools.partial(
        pl.kernel, mesh=mesh,
        out_type=jax.ShapeDtypeStruct((NB * N * D,), jnp.float32),
        scratch_types=[
            pltpu.VMEM((K_ALL * (D // 2),), jnp.int32),
            pltpu.VMEM((CHUNK * D,), jnp.float32),
            pltpu.VMEM((CHUNK * MS,), jnp.int32),
            pltpu.VMEM((CHUNK * MS,), jnp.float32),
        ],
        compiler_params=pltpu.CompilerParams(needs_layout_passes=False),
    )(_sc_gather_body)
    return f(pbp_flat, idx_flat, w_flat)


def _final_body(g_in_ref, q_ref, e_ref, bu_ref, amat_ref, umat_ref,
                lng_ref, lnb_ref, qn_ref):
    q = q_ref[0]
    g = g_in_ref[0]
    Qd = D // 4
    msg = bu_ref[...][None, :]
    for a in range(4):
        qa = q[:, a*Qd:(a+1)*Qd]
        qa4 = jnp.concatenate([qa, qa, qa, qa], axis=1)
        msg = msg + jnp.dot(qa4 * g, e_ref[a], preferred_element_type=jnp.float32)
    x = q + msg
    A = amat_ref[...]
    U = umat_ref[...]
    mu = jnp.dot(x, A, preferred_element_type=jnp.float32)
    m2 = jnp.dot(x * x, A, preferred_element_type=jnp.float32)
    rs = lax.rsqrt(m2 - mu * mu + 1e-5)
    mu_e = jnp.dot(mu, U, preferred_element_type=jnp.float32)
    rs_e = jnp.dot(rs, U, preferred_element_type=jnp.float32)
    qn_ref[0] = (x - mu_e) * rs_e * lng_ref[...][None, :] + lnb_ref[...][None, :]


def _final_call(g, q, e_stack, bu, amat, umat, gvec, bvec):
    return pl.pallas_call(
        _final_body,
        grid=(NB, NT),
        in_specs=[
            pl.BlockSpec((1, TM, D), lambda b, t: (b, t, 0)),
            pl.BlockSpec((1, TM, D), lambda b, t: (b, t, 0)),
            pl.BlockSpec((4, D, D), lambda b, t: (0, 0, 0)),
            pl.BlockSpec((D,), lambda b, t: (0,)),
            pl.BlockSpec((D, 8), lambda b, t: (0, 0)),
            pl.BlockSpec((8, D), lambda b, t: (0, 0)),
            pl.BlockSpec((D,), lambda b, t: (0,)),
            pl.BlockSpec((D,), lambda b, t: (0,)),
        ],
        out_specs=pl.BlockSpec((1, TM, D), lambda b, t: (b, t, 0)),
        out_shape=jax.ShapeDtypeStruct((NB, N, D), jnp.float32),
    )(g, q, e_stack, bu, amat, umat, gvec, bvec)


def kernel(q, idx_t, w_t, idx_v, w_v, idx_e, w_e, time_codes, var_codes,
           event_codes, contribute_mask, params):
    idx_all = jnp.concatenate(
        [idx_t, idx_v + KT, idx_e + (KT + KV)], axis=2).astype(jnp.int32)
    w_all = jnp.concatenate([w_t, w_v, w_e], axis=2) * contribute_mask[:, :, None]
    codes_all = jnp.concatenate([time_codes, var_codes, event_codes], axis=0)

    wq, bq = zip(*(_assemble_qlin(params[k]) for k in ('proto_t', 'proto_v', 'proto_e')))
    wq_stack = jnp.stack(wq)
    bq_stack = jnp.stack(bq)
    wu, bu = _assemble_qlin(params['update_proj'])
    gvec = jnp.concatenate(params['ln_g'])
    bvec = jnp.concatenate(params['ln_b'])

    Qd = D // 4
    ham = [[(0, 1), (1, 1), (2, 1), (3, 1)],
           [(1, 1), (0, -1), (3, 1), (2, -1)],
           [(2, 1), (3, -1), (0, -1), (1, 1)],
           [(3, 1), (2, 1), (1, -1), (0, -1)]]
    e_stack = jnp.stack([
        jnp.concatenate([s * wu[c*Qd:(c+1)*Qd, :] for (c, s) in ham[a]], axis=0)
        for a in range(4)])
    lane = jnp.arange(D)
    amat = ((lane[:, None] // Qd) == jnp.arange(8)[None, :]).astype(jnp.float32) / Qd
    umat = (jnp.arange(8)[:, None] == (lane[None, :] // Qd)).astype(jnp.float32)

    pbs, qns = [], []
    halves = []
    for h in range(B // NB):
        sl = slice(h * NB, (h + 1) * NB)
        q_h = q[sl]
        idx_h = idx_all[sl].reshape(-1)
        w_h = w_all[sl].reshape(-1)
        parts = _sc_scatter(q_h.reshape(-1), idx_h, w_h)
        halves.append((q_h, idx_h, w_h, parts))
    for q_h, idx_h, w_h, parts in halves:
        pb_h = _mid_call(parts.reshape(NB * WPB, K_ALL, RW),
                         codes_all, wq_stack, bq_stack)
        pbp_h = jax.lax.bitcast_convert_type(
            pb_h.astype(jnp.bfloat16).reshape(NB, K_ALL, D // 2, 2), jnp.int32)
        g_h = _sc_gather(pbp_h.reshape(-1), idx_h, w_h).reshape(NB, N, D)
        qn_h = _final_call(g_h, q_h, e_stack, bu, amat, umat, gvec, bvec)
        pbs.append(pb_h)
        qns.append(qn_h)

    pb = jnp.concatenate(pbs, axis=0)
    q_new = jnp.concatenate(qns, axis=0)
    proto_t = pb[:, :KT, :]
    proto_v = pb[:, KT:KT+KV, :]
    proto_e = pb[:, KT+KV:, :]
    return (q_new, proto_t, proto_v, proto_e)


# speedup vs baseline: 1.0552x; 1.0552x over previous
import functools
import jax
import jax.numpy as jnp
from jax import lax
from jax.experimental import pallas as pl
from jax.experimental.pallas import tpu as pltpu
from jax.experimental.pallas import tpu_sc as plsc

B, N, M, D = 8, 2048, 8, 128
KT, KV, KE = 64, 128, 256
K_ALL = KT + KV + KE
MS = 3 * M
RW = D + 16
TM = 128
NT = N // TM
NWORK = 32
TPW = (B * N) // NWORK
CHUNK = 128
NCH = TPW // CHUNK


def _assemble_qlin(p):
    r, i, j, k = p['r'], p['i'], p['j'], p['k']
    W = jnp.concatenate([
        jnp.concatenate([r, -i, -j, -k], axis=1),
        jnp.concatenate([i, r, -k, j], axis=1),
        jnp.concatenate([j, k, r, -i], axis=1),
        jnp.concatenate([k, -j, i, r], axis=1)], axis=0)
    return W.T, p['b']


NB = 4
TPW2 = (NB * N) // NWORK
NCH2 = TPW2 // CHUNK
WPB = NWORK // NB


def _sc_scatter_body(q_hbm, idx_hbm, w_hbm, out_hbm, accf, qbuf, ibuf, wbuf):
    wid = lax.axis_index("c") * 16 + lax.axis_index("s")
    tok0 = wid * TPW2
    iota = lax.iota(jnp.int32, 16)
    lane0 = iota == 0

    @plsc.parallel_loop(0, (K_ALL * RW) // 16, unroll=8)
    def _zero(i):
        accf[pl.ds(i * 16, 16)] = jnp.zeros((16,), jnp.float32)

    for ch in range(NCH2):
        base = tok0 + ch * CHUNK
        pltpu.sync_copy(q_hbm.at[pl.ds(base * D, CHUNK * D)], qbuf)
        pltpu.sync_copy(idx_hbm.at[pl.ds(base * MS, CHUNK * MS)], ibuf)
        pltpu.sync_copy(w_hbm.at[pl.ds(base * MS, CHUNK * MS)], wbuf)

        @plsc.parallel_loop(0, CHUNK, unroll=2)
        def _tok(t):
            qv = [qbuf[pl.ds(t * D + cb * 16, 16)] for cb in range(D // 16)]
            for j in range(MS):
                sp = jnp.full((16,), t * MS + j, jnp.int32)
                rowv = plsc.load_gather(ibuf, [sp])
                wv = plsc.load_gather(wbuf, [sp])
                addr = rowv * RW + iota
                for cb in range(D // 16):
                    plsc.addupdate_scatter(accf, [addr + cb * 16], qv[cb] * wv)
                plsc.addupdate_scatter(accf, [addr + D], wv, mask=lane0)

    pltpu.sync_copy(accf, out_hbm.at[pl.ds(wid * K_ALL * RW, K_ALL * RW)])


def _sc_scatter(q_half, idx_half, w_half):
    mesh = plsc.VectorSubcoreMesh(core_axis_name="c", subcore_axis_name="s")
    f = functools.partial(
        pl.kernel, mesh=mesh,
        out_type=jax.ShapeDtypeStruct((NWORK * K_ALL * RW,), jnp.float32),
        scratch_types=[
            pltpu.VMEM((K_ALL * RW,), jnp.float32),
            pltpu.VMEM((CHUNK * D,), jnp.float32),
            pltpu.VMEM((CHUNK * MS,), jnp.int32),
            pltpu.VMEM((CHUNK * MS,), jnp.float32),
        ],
        compiler_params=pltpu.CompilerParams(needs_layout_passes=False),
    )(_sc_scatter_body)
    return f(q_half, idx_half, w_half)


def _mid_body(parts_ref, codes_ref, wq_ref, bq_ref, pb_ref):
    psum = jnp.sum(parts_ref[...], axis=0)
    proto = psum[:, :D]
    ws = psum[:, D:D+1]
    codes = codes_ref[...]
    wsc = jnp.maximum(ws, 0.001)
    blend = jnp.clip(ws / (ws + 0.5), 0.0, 1.0)
    pb = blend * (proto / wsc) + (1.0 - blend) * codes

    outs = []
    offs = [0, KT, KT + KV, K_ALL]
    for c in range(3):
        seg = pb[offs[c]:offs[c+1], :]
        outs.append(jnp.dot(seg, wq_ref[c], preferred_element_type=jnp.float32)
                    + bq_ref[c][None, :])
    pb_ref[0] = jnp.concatenate(outs, axis=0)


def _mid_call(parts, codes_all, wq_stack, bq_stack):
    return pl.pallas_call(
        _mid_body,
        grid=(NB,),
        in_specs=[
            pl.BlockSpec((WPB, K_ALL, RW), lambda b: (b, 0, 0)),
            pl.BlockSpec((K_ALL, D), lambda b: (0, 0)),
            pl.BlockSpec((3, D, D), lambda b: (0, 0, 0)),
            pl.BlockSpec((3, D), lambda b: (0, 0)),
        ],
        out_specs=pl.BlockSpec((1, K_ALL, D), lambda b: (b, 0, 0)),
        out_shape=jax.ShapeDtypeStruct((NB, K_ALL, D), jnp.float32),
    )(parts, codes_all, wq_stack, bq_stack)


def _sc_gather_body(pb_hbm, idx_hbm, w_hbm, g_hbm, tbuf, gbuf, ibuf, wbuf):
    wid = lax.axis_index("c") * 16 + lax.axis_index("s")
    b = wid // WPB
    tok0 = wid * TPW2
    iota = lax.iota(jnp.int32, 16)
    iota2 = iota * 2
    DH = D // 2

    pltpu.sync_copy(pb_hbm.at[pl.ds(b * K_ALL * DH, K_ALL * DH)], tbuf)

    for ch in range(NCH2):
        base = tok0 + ch * CHUNK
        pltpu.sync_copy(idx_hbm.at[pl.ds(base * MS, CHUNK * MS)], ibuf)
        pltpu.sync_copy(w_hbm.at[pl.ds(base * MS, CHUNK * MS)], wbuf)

        def _tok(t, _):
            acc_e = [jnp.zeros((16,), jnp.float32) for _ in range(DH // 16)]
            acc_o = [jnp.zeros((16,), jnp.float32) for _ in range(DH // 16)]
            for j in range(MS):
                sp = jnp.full((16,), t * MS + j, jnp.int32)
                rowv = plsc.load_gather(ibuf, [sp])
                wv = plsc.load_gather(wbuf, [sp])
                addr = rowv * DH + iota
                for cb in range(DH // 16):
                    v = plsc.load_gather(tbuf, [addr + cb * 16])
                    ev, ov = plsc.unpack(plsc.bitcast(v, jnp.bfloat16),
                                         format=plsc.PackFormat.INTERLEAVED)
                    acc_e[cb] = acc_e[cb] + ev * wv
                    acc_o[cb] = acc_o[cb] + ov * wv
            for cb in range(DH // 16):
                sbase = jnp.full((16,), t * D + cb * 32, jnp.int32) + iota2
                plsc.store_scatter(gbuf, [sbase], acc_e[cb])
                plsc.store_scatter(gbuf, [sbase + 1], acc_o[cb])
            return _
        lax.fori_loop(0, CHUNK, _tok, None)

        pltpu.sync_copy(gbuf, g_hbm.at[pl.ds(base * D, CHUNK * D)])


def _sc_gather(pbp_flat, idx_flat, w_flat):
    mesh = plsc.VectorSubcoreMesh(core_axis_name="c", subcore_axis_name="s")
    f = functools.partial(
        pl.kernel, mesh=mesh,
        out_type=jax.ShapeDtypeStruct((NB * N * D,), jnp.float32),
        scratch_types=[
            pltpu.VMEM((K_ALL * (D // 2),), jnp.int32),
            pltpu.VMEM((CHUNK * D,), jnp.float32),
            pltpu.VMEM((CHUNK * MS,), jnp.int32),
            pltpu.VMEM((CHUNK * MS,), jnp.float32),
        ],
        compiler_params=pltpu.CompilerParams(needs_layout_passes=False),
    )(_sc_gather_body)
    return f(pbp_flat, idx_flat, w_flat)


def _final_body(g_in_ref, q_ref, e_ref, bu_ref, amat_ref, umat_ref,
                lng_ref, lnb_ref, qn_ref):
    q = q_ref[0]
    g = g_in_ref[0]
    Qd = D // 4
    msg = bu_ref[...][None, :]
    for a in range(4):
        qa = q[:, a*Qd:(a+1)*Qd]
        qa4 = jnp.concatenate([qa, qa, qa, qa], axis=1)
        msg = msg + jnp.dot(qa4 * g, e_ref[a], preferred_element_type=jnp.float32)
    x = q + msg
    A = amat_ref[...]
    U = umat_ref[...]
    mu = jnp.dot(x, A, preferred_element_type=jnp.float32)
    m2 = jnp.dot(x * x, A, preferred_element_type=jnp.float32)
    rs = lax.rsqrt(m2 - mu * mu + 1e-5)
    mu_e = jnp.dot(mu, U, preferred_element_type=jnp.float32)
    rs_e = jnp.dot(rs, U, preferred_element_type=jnp.float32)
    qn_ref[0] = (x - mu_e) * rs_e * lng_ref[...][None, :] + lnb_ref[...][None, :]


def _final_call(g, q, e_stack, bu, amat, umat, gvec, bvec):
    return pl.pallas_call(
        _final_body,
        grid=(NB, NT),
        in_specs=[
            pl.BlockSpec((1, TM, D), lambda b, t: (b, t, 0)),
            pl.BlockSpec((1, TM, D), lambda b, t: (b, t, 0)),
            pl.BlockSpec((4, D, D), lambda b, t: (0, 0, 0)),
            pl.BlockSpec((D,), lambda b, t: (0,)),
            pl.BlockSpec((D, 8), lambda b, t: (0, 0)),
            pl.BlockSpec((8, D), lambda b, t: (0, 0)),
            pl.BlockSpec((D,), lambda b, t: (0,)),
            pl.BlockSpec((D,), lambda b, t: (0,)),
        ],
        out_specs=pl.BlockSpec((1, TM, D), lambda b, t: (b, t, 0)),
        out_shape=jax.ShapeDtypeStruct((NB, N, D), jnp.float32),
    )(g, q, e_stack, bu, amat, umat, gvec, bvec)


def kernel(q, idx_t, w_t, idx_v, w_v, idx_e, w_e, time_codes, var_codes,
           event_codes, contribute_mask, params):
    idx_all = jnp.concatenate(
        [idx_t, idx_v + KT, idx_e + (KT + KV)], axis=2).astype(jnp.int32)
    w_all = jnp.concatenate([w_t, w_v, w_e], axis=2) * contribute_mask[:, :, None]
    codes_all = jnp.concatenate([time_codes, var_codes, event_codes], axis=0)

    wq, bq = zip(*(_assemble_qlin(params[k]) for k in ('proto_t', 'proto_v', 'proto_e')))
    wq_stack = jnp.stack(wq)
    bq_stack = jnp.stack(bq)
    wu, bu = _assemble_qlin(params['update_proj'])
    gvec = jnp.concatenate(params['ln_g'])
    bvec = jnp.concatenate(params['ln_b'])

    Qd = D // 4
    ham = [[(0, 1), (1, 1), (2, 1), (3, 1)],
           [(1, 1), (0, -1), (3, 1), (2, -1)],
           [(2, 1), (3, -1), (0, -1), (1, 1)],
           [(3, 1), (2, 1), (1, -1), (0, -1)]]
    e_stack = jnp.stack([
        jnp.concatenate([s * wu[c*Qd:(c+1)*Qd, :] for (c, s) in ham[a]], axis=0)
        for a in range(4)])
    lane = jnp.arange(D)
    amat = ((lane[:, None] // Qd) == jnp.arange(8)[None, :]).astype(jnp.float32) / Qd
    umat = (jnp.arange(8)[:, None] == (lane[None, :] // Qd)).astype(jnp.float32)

    pbs, qns = [], []
    halves = []
    for h in range(B // NB):
        sl = slice(h * NB, (h + 1) * NB)
        q_h = q[sl]
        idx_h = idx_all[sl].reshape(-1)
        w_h = w_all[sl].reshape(-1)
        parts = _sc_scatter(q_h.reshape(-1), idx_h, w_h)
        halves.append((q_h, idx_h, w_h, parts))
    for q_h, idx_h, w_h, parts in halves:
        pb_h = _mid_call(parts.reshape(NB * WPB, K_ALL, RW),
                         codes_all, wq_stack, bq_stack)
        pbp_h = jax.lax.bitcast_convert_type(
            pb_h.astype(jnp.bfloat16).reshape(NB, K_ALL, D // 2, 2), jnp.int32)
        g_h = _sc_gather(pbp_h.reshape(-1), idx_h, w_h).reshape(NB, N, D)
        qn_h = _final_call(g_h, q_h, e_stack, bu, amat, umat, gvec, bvec)
        pbs.append(pb_h)
        qns.append(qn_h)

    pb = jnp.concatenate(pbs, axis=0)
    q_new = jnp.concatenate(qns, axis=0)
    proto_t = pb[:, :KT, :]
    proto_v = pb[:, KT:KT+KV, :]
    proto_e = pb[:, KT+KV:, :]
    return (q_new, proto_t, proto_v, proto_e)
